# baseline (speedup 1.0000x reference)
import jax
import jax.numpy as jnp
from jax.experimental import pallas as pl
from jax.experimental.pallas import tpu as pltpu


def _rnn_kernel(markers_ref, data_ref, Wr_ref, hid_ref, Wout_ref,
                bout_ref, out_ref, hout_ref, g_ref, i_ref):
    l = pl.program_id(0)
    nlayers = pl.num_programs(0)
    hid = hid_ref.shape[1]

    @pl.when(l == 0)
    def _init():
        i_ref[0] = 0
        g_ref[0:1, 0:hid] = hid_ref[...]
        g_ref[0:1, hid + 1:hid + 2] = jnp.ones((1, 1), jnp.float32)

    cnt = markers_ref[0, l]

    def body(_, carry):
        i = i_ref[0]
        g_ref[0:1, hid:hid + 1] = jnp.full((1, 1), data_ref[0, i], jnp.float32)
        t = jax.lax.dot_general(
            g_ref[...], Wr_ref[0],
            (((1,), (0,)), ((), ())),
            preferred_element_type=jnp.float32)
        g_ref[0:1, 0:hid] = jnp.maximum(t, 0.0)
        i_ref[0] = i + 1
        return carry

    jax.lax.fori_loop(0, cnt, body, 0, unroll=False)

    @pl.when(l == nlayers - 1)
    def _finish():
        h_fin = g_ref[0:1, 0:hid]
        logits = jax.lax.dot_general(
            h_fin, Wout_ref[...],
            (((1,), (0,)), ((), ())),
            preferred_element_type=jnp.float32) + bout_ref[...]
        m = jnp.max(logits)
        z = logits - m
        out_ref[...] = z - jnp.log(jnp.sum(jnp.exp(z)))
        hout_ref[...] = h_fin


@jax.jit
def kernel(input_data, markers, hidden, W, b, W_out, b_out):
    nlayers, hid, inpp1 = W.shape
    out_sz = W_out.shape[0]
    aug = hid + 2

    Wt = jnp.transpose(W, (0, 2, 1))
    Wr = jnp.concatenate([Wt[:, 1:, :], Wt[:, 0:1, :], b[:, None, :]], axis=1)
    Wout_t = W_out.T
    bout_row = b_out[None, :]

    out_row, h_row = pl.pallas_call(
        _rnn_kernel,
        grid=(nlayers,),
        in_specs=[
            pl.BlockSpec(memory_space=pltpu.SMEM),
            pl.BlockSpec(memory_space=pltpu.SMEM),
            pl.BlockSpec((1, aug, hid), lambda l: (l, 0, 0)),
            pl.BlockSpec((1, hid), lambda l: (0, 0)),
            pl.BlockSpec((hid, out_sz), lambda l: (0, 0)),
            pl.BlockSpec((1, out_sz), lambda l: (0, 0)),
        ],
        out_specs=[
            pl.BlockSpec((1, out_sz), lambda l: (0, 0)),
            pl.BlockSpec((1, hid), lambda l: (0, 0)),
        ],
        out_shape=[
            jax.ShapeDtypeStruct((1, out_sz), jnp.float32),
            jax.ShapeDtypeStruct((1, hid), jnp.float32),
        ],
        scratch_shapes=[
            pltpu.VMEM((1, aug), jnp.float32),
            pltpu.SMEM((1,), jnp.int32),
        ],
    )(markers, input_data, Wr, hidden, Wout_t, bout_row)

    return out_row, h_row


# speedup vs baseline: 1.4394x; 1.4394x over previous
import jax
import jax.numpy as jnp
from jax.experimental import pallas as pl
from jax.experimental.pallas import tpu as pltpu


def _rnn_kernel(markers_ref, data_ref, W_ref, b_ref, hid_ref, Wout_ref,
                bout_ref, out_ref, hout_ref, g_ref, i_ref):
    l = pl.program_id(0)
    nlayers = pl.num_programs(0)

    @pl.when(l == 0)
    def _init():
        i_ref[0] = 0
        g_ref[0:1, 1:1025] = hid_ref[...]

    cnt = markers_ref[0, l]

    def step():
        i = i_ref[0]
        g_ref[0:1, 0:1] = jnp.full((1, 1), data_ref[0, i], jnp.float32)
        t = jax.lax.dot_general(
            g_ref[...], W_ref[...],
            (((1,), (1,)), ((), ())),
            preferred_element_type=jnp.float32)
        h_new = jnp.maximum(t + b_ref[0], 0.0)
        g_ref[0:1, 1:1025] = h_new
        i_ref[0] = i + 1

    def body(_, carry):
        step()
        return carry

    jax.lax.fori_loop(0, cnt, body, 0, unroll=False)

    @pl.when(l == nlayers - 1)
    def _finish():
        h_fin = g_ref[0:1, 1:1025]
        logits = jax.lax.dot_general(
            h_fin, Wout_ref[...],
            (((1,), (1,)), ((), ())),
            preferred_element_type=jnp.float32) + bout_ref[...]
        m = jnp.max(logits)
        z = logits - m
        out_ref[...] = z - jnp.log(jnp.sum(jnp.exp(z)))
        hout_ref[...] = h_fin


@jax.jit
def kernel(input_data, markers, hidden, W, b, W_out, b_out):
    nlayers, hid, inpp1 = W.shape
    out_sz = W_out.shape[0]

    b_row = b[:, None, :]
    bout_row = b_out[None, :]

    out_row, h_row = pl.pallas_call(
        _rnn_kernel,
        grid=(nlayers,),
        in_specs=[
            pl.BlockSpec(memory_space=pltpu.SMEM),
            pl.BlockSpec(memory_space=pltpu.SMEM),
            pl.BlockSpec((hid, inpp1), lambda l: (l, 0)),
            pl.BlockSpec((1, 1, hid), lambda l: (l, 0, 0)),
            pl.BlockSpec((1, hid), lambda l: (0, 0)),
            pl.BlockSpec((out_sz, hid), lambda l: (0, 0)),
            pl.BlockSpec((1, out_sz), lambda l: (0, 0)),
        ],
        out_specs=[
            pl.BlockSpec((1, out_sz), lambda l: (0, 0)),
            pl.BlockSpec((1, hid), lambda l: (0, 0)),
        ],
        out_shape=[
            jax.ShapeDtypeStruct((1, out_sz), jnp.float32),
            jax.ShapeDtypeStruct((1, hid), jnp.float32),
        ],
        scratch_shapes=[
            pltpu.VMEM((1, inpp1), jnp.float32),
            pltpu.SMEM((1,), jnp.int32),
        ],
    )(markers, input_data, W.reshape(nlayers * hid, inpp1), b_row, hidden,
      W_out, bout_row)

    return out_row, h_row


# speedup vs baseline: 1.4533x; 1.0096x over previous
import jax
import jax.numpy as jnp
from jax.experimental import pallas as pl
from jax.experimental.pallas import tpu as pltpu


def _rnn_kernel(markers_ref, data_ref, W_ref, b_ref, hid_ref, Wout_ref,
                bout_ref, out_ref, hout_ref, g_ref, i_ref):
    l = pl.program_id(0)
    nlayers = pl.num_programs(0)

    @pl.when(l == 0)
    def _init():
        i_ref[0] = 0
        g_ref[0:1, 1:1025] = hid_ref[...]

    cnt = markers_ref[0, l]

    def step():
        i = i_ref[0]
        g_ref[0:1, 0:1] = jnp.full((1, 1), data_ref[0, i], jnp.float32)
        t = jax.lax.dot_general(
            g_ref[...], W_ref[0],
            (((1,), (1,)), ((), ())),
            preferred_element_type=jnp.float32)
        h_new = jnp.maximum(t + b_ref[0], 0.0)
        g_ref[0:1, 1:1025] = h_new
        i_ref[0] = i + 1

    def body(_, carry):
        step()
        return carry

    jax.lax.fori_loop(0, cnt, body, 0, unroll=False)

    @pl.when(l == nlayers - 1)
    def _finish():
        h_fin = g_ref[0:1, 1:1025]
        logits = jax.lax.dot_general(
            h_fin, Wout_ref[...],
            (((1,), (1,)), ((), ())),
            preferred_element_type=jnp.float32) + bout_ref[...]
        m = jnp.max(logits)
        z = logits - m
        out_ref[...] = z - jnp.log(jnp.sum(jnp.exp(z)))
        hout_ref[...] = h_fin


@jax.jit
def kernel(input_data, markers, hidden, W, b, W_out, b_out):
    nlayers, hid, inpp1 = W.shape
    out_sz = W_out.shape[0]

    b_row = b[:, None, :]
    bout_row = b_out[None, :]

    out_row, h_row = pl.pallas_call(
        _rnn_kernel,
        grid=(nlayers,),
        in_specs=[
            pl.BlockSpec(memory_space=pltpu.SMEM),
            pl.BlockSpec(memory_space=pltpu.SMEM),
            pl.BlockSpec((1, hid, inpp1), lambda l: (l, 0, 0)),
            pl.BlockSpec((1, 1, hid), lambda l: (l, 0, 0)),
            pl.BlockSpec((1, hid), lambda l: (0, 0)),
            pl.BlockSpec((out_sz, hid), lambda l: (0, 0)),
            pl.BlockSpec((1, out_sz), lambda l: (0, 0)),
        ],
        out_specs=[
            pl.BlockSpec((1, out_sz), lambda l: (0, 0)),
            pl.BlockSpec((1, hid), lambda l: (0, 0)),
        ],
        out_shape=[
            jax.ShapeDtypeStruct((1, out_sz), jnp.float32),
            jax.ShapeDtypeStruct((1, hid), jnp.float32),
        ],
        scratch_shapes=[
            pltpu.VMEM((1, inpp1), jnp.float32),
            pltpu.SMEM((1,), jnp.int32),
        ],
    )(markers, input_data, W, b_row, hidden,
      W_out, bout_row)

    return out_row, h_row
